# baseline (speedup 1.0000x reference)
import functools

import jax
import jax.numpy as jnp
from jax import lax
from jax.experimental import pallas as pl
from jax.experimental.pallas import tpu as pltpu
from jax.experimental.pallas import tpu_sc as plsc

_NC = 2
_NS = 16
_NW = _NC * _NS
_CHUNK = 128
_GROUP = 10


_NBUF = 4
_SUB = 2048


def _mlp(emb, w1_ref, b1_ref, w2t_ref, b2_ref):
    h = jnp.dot(emb, w1_ref[...], preferred_element_type=jnp.float32)
    h = jnp.maximum(h + b1_ref[...], 0.0)
    ot = lax.dot_general(w2t_ref[...], h,
                         dimension_numbers=(((1,), (1,)), ((), ())),
                         preferred_element_type=jnp.float32)
    return ot + b2_ref[...]


def _mlp_body(t_hbm, w1_ref, b1_ref, w2t_ref, b2_ref, o0_ref, o1_ref,
              tbuf, tailbuf, sems, tailsem):
    v = t_hbm.shape[0]
    n_full = v // _SUB
    tail = v - n_full * _SUB
    tail0 = n_full * _SUB

    def dma(j, slot):
        return pltpu.make_async_copy(
            t_hbm.at[pl.ds(j * _SUB, _SUB), :], tbuf.at[slot], sems.at[slot])

    tail_dma = pltpu.make_async_copy(
        t_hbm.at[pl.ds(tail0, tail), :], tailbuf, tailsem)
    tail_dma.start()
    for j in range(_NBUF):
        dma(j, j).start()

    def step(j, carry):
        slot = lax.rem(j, _NBUF)
        dma(j, slot).wait()
        ot = _mlp(tbuf[slot], w1_ref, b1_ref, w2t_ref, b2_ref)
        o0_ref[pl.ds(j * _SUB, _SUB)] = ot[0:1, :].reshape(_SUB)
        o1_ref[pl.ds(j * _SUB, _SUB)] = ot[1:2, :].reshape(_SUB)

        @pl.when(j + _NBUF < n_full)
        def _():
            dma(j + _NBUF, slot).start()

        return carry

    lax.fori_loop(0, n_full, step, 0)
    tail_dma.wait()
    ot = _mlp(tailbuf[...], w1_ref, b1_ref, w2t_ref, b2_ref)
    o0_ref[pl.ds(tail0, tail)] = ot[0:1, :].reshape(tail)
    o1_ref[pl.ds(tail0, tail)] = ot[1:2, :].reshape(tail)


def _fuse_table(table, W1, b1, W2, b2):
    v, d = table.shape
    dh = W1.shape[1]
    do = W2.shape[1]
    out1d = jax.ShapeDtypeStruct((v,), jnp.float32)
    return pl.pallas_call(
        _mlp_body,
        in_specs=[
            pl.BlockSpec(memory_space=pl.ANY),
            pl.BlockSpec((d, dh), lambda: (0, 0)),
            pl.BlockSpec((1, dh), lambda: (0, 0)),
            pl.BlockSpec((do, dh), lambda: (0, 0)),
            pl.BlockSpec((do, 1), lambda: (0, 0)),
        ],
        out_specs=[
            pl.BlockSpec((v,), lambda: (0,)),
            pl.BlockSpec((v,), lambda: (0,)),
        ],
        out_shape=[out1d, out1d],
        scratch_shapes=[
            pltpu.VMEM((_NBUF, _SUB, d), jnp.float32),
            pltpu.VMEM((v - (v // _SUB) * _SUB, d), jnp.float32),
            pltpu.SemaphoreType.DMA((_NBUF,)),
            pltpu.SemaphoreType.DMA,
        ],
    )(table, W1, b1.reshape(1, dh), W2.T, b2.reshape(do, 1))


def _gather_rows(idx1d, f0, f1):
    n_idx = idx1d.shape[0]
    chunk = _CHUNK
    n_chunks = n_idx // chunk
    per_tile = n_chunks // _NW
    n_groups = per_tile // _GROUP
    npt = per_tile * chunk
    mesh = plsc.VectorSubcoreMesh(core_axis_name="c", subcore_axis_name="s")
    out1d = jax.ShapeDtypeStruct((n_idx,), jnp.float32)

    @functools.partial(
        pl.kernel,
        out_type=[out1d, out1d],
        mesh=mesh,
        scratch_types=[
            pltpu.VMEM((npt,), jnp.int32),
            pltpu.VMEM((npt,), jnp.float32),
            pltpu.VMEM((npt,), jnp.float32),
            pltpu.SemaphoreType.DMA,
        ],
        compiler_params=pltpu.CompilerParams(use_tc_tiling_on_sc=False),
    )
    def gather_kernel(idx_hbm, f0_hbm, f1_hbm, o0_hbm, o1_hbm,
                      idx_v, g0_v, g1_v, sem):
        wid = lax.axis_index("s") * _NC + lax.axis_index("c")
        base = wid * npt
        pltpu.sync_copy(idx_hbm.at[pl.ds(base, npt)], idx_v)

        def group(g, carry):
            handles = []
            for u in range(_GROUP):
                j = g * _GROUP + u
                sl = pl.ds(j * chunk, chunk)
                handles.append(pltpu.async_copy(
                    f0_hbm.at[idx_v.at[sl]], g0_v.at[sl], sem))
                handles.append(pltpu.async_copy(
                    f1_hbm.at[idx_v.at[sl]], g1_v.at[sl], sem))
            for h in handles:
                h.wait()
            return carry

        lax.fori_loop(0, n_groups, group, 0)
        pltpu.sync_copy(g0_v, o0_hbm.at[pl.ds(base, npt)])
        pltpu.sync_copy(g1_v, o1_hbm.at[pl.ds(base, npt)])

    return gather_kernel(idx1d, f0, f1)


def kernel(x, table, W1, b1, W2, b2):
    b, s = x.shape
    f0, f1 = _fuse_table(table, W1, b1, W2, b2)
    idx1d = x.astype(jnp.int32).reshape(-1)
    o0, o1 = _gather_rows(idx1d, f0, f1)
    return jnp.stack([o0, o1], axis=-1).reshape(b, s, W2.shape[1])


# speedup vs baseline: 1.0158x; 1.0158x over previous
import functools

import jax
import jax.numpy as jnp
from jax import lax
from jax.experimental import pallas as pl
from jax.experimental.pallas import tpu as pltpu
from jax.experimental.pallas import tpu_sc as plsc

_NC = 2
_NS = 16
_NW = _NC * _NS
_CHUNK = 128
_GROUP = 10


_NBUF = 4
_SUB = 2048


def _mlp(emb, w1_ref, b1_ref, w2t_ref, b2_ref):
    h = jnp.dot(emb, w1_ref[...], preferred_element_type=jnp.float32)
    h = jnp.maximum(h + b1_ref[...], 0.0)
    ot = lax.dot_general(w2t_ref[...], h,
                         dimension_numbers=(((1,), (1,)), ((), ())),
                         preferred_element_type=jnp.float32)
    return ot + b2_ref[...]


def _mlp_body(t_hbm, w1_ref, b1_ref, w2t_ref, b2_ref, o0_ref, o1_ref,
              tbuf, tailbuf, sems, tailsem):
    v = t_hbm.shape[0]
    n_full = v // _SUB
    tail = v - n_full * _SUB
    tail0 = n_full * _SUB

    def dma(j, slot):
        return pltpu.make_async_copy(
            t_hbm.at[pl.ds(j * _SUB, _SUB), :], tbuf.at[slot], sems.at[slot])

    tail_dma = pltpu.make_async_copy(
        t_hbm.at[pl.ds(tail0, tail), :], tailbuf, tailsem)
    tail_dma.start()
    for j in range(_NBUF):
        dma(j, j).start()

    def step(j, carry):
        slot = lax.rem(j, _NBUF)
        dma(j, slot).wait()
        ot = _mlp(tbuf[slot], w1_ref, b1_ref, w2t_ref, b2_ref)
        o0_ref[pl.ds(j * _SUB, _SUB)] = ot[0:1, :].reshape(_SUB)
        o1_ref[pl.ds(j * _SUB, _SUB)] = ot[1:2, :].reshape(_SUB)

        @pl.when(j + _NBUF < n_full)
        def _():
            dma(j + _NBUF, slot).start()

        return carry

    lax.fori_loop(0, n_full, step, 0)
    tail_dma.wait()
    ot = _mlp(tailbuf[...], w1_ref, b1_ref, w2t_ref, b2_ref)
    o0_ref[pl.ds(tail0, tail)] = ot[0:1, :].reshape(tail)
    o1_ref[pl.ds(tail0, tail)] = ot[1:2, :].reshape(tail)


def _fuse_table(table, W1, b1, W2, b2):
    v, d = table.shape
    dh = W1.shape[1]
    do = W2.shape[1]
    out1d = jax.ShapeDtypeStruct((v,), jnp.float32)
    return pl.pallas_call(
        _mlp_body,
        in_specs=[
            pl.BlockSpec(memory_space=pl.ANY),
            pl.BlockSpec((d, dh), lambda: (0, 0)),
            pl.BlockSpec((1, dh), lambda: (0, 0)),
            pl.BlockSpec((do, dh), lambda: (0, 0)),
            pl.BlockSpec((do, 1), lambda: (0, 0)),
        ],
        out_specs=[
            pl.BlockSpec((v,), lambda: (0,)),
            pl.BlockSpec((v,), lambda: (0,)),
        ],
        out_shape=[out1d, out1d],
        scratch_shapes=[
            pltpu.VMEM((_NBUF, _SUB, d), jnp.float32),
            pltpu.VMEM((v - (v // _SUB) * _SUB, d), jnp.float32),
            pltpu.SemaphoreType.DMA((_NBUF,)),
            pltpu.SemaphoreType.DMA,
        ],
    )(table, W1, b1.reshape(1, dh), W2.T, b2.reshape(do, 1))


def _gather_rows(idx1d, f0, f1):
    n_idx = idx1d.shape[0]
    v = f0.shape[0]
    per_tile = n_idx // _NS
    lanes = 16
    mesh = plsc.VectorSubcoreMesh(core_axis_name="c", subcore_axis_name="s")
    out1d = jax.ShapeDtypeStruct((n_idx,), jnp.float32)

    @functools.partial(
        pl.kernel,
        out_type=[out1d, out1d],
        mesh=mesh,
        scratch_types=[
            pltpu.VMEM((v,), jnp.float32),
            pltpu.VMEM((per_tile,), jnp.int32),
            pltpu.VMEM((per_tile,), jnp.float32),
            pltpu.SemaphoreType.DMA,
            pltpu.SemaphoreType.DMA,
        ],
        compiler_params=pltpu.CompilerParams(use_tc_tiling_on_sc=False, needs_layout_passes=False),
    )
    def gather_kernel(idx_hbm, f0_hbm, f1_hbm, o0_hbm, o1_hbm,
                      ftab_v, idx_v, vals_v, sem_t, sem_i):
        col = lax.axis_index("c")
        tile = lax.axis_index("s")
        base = tile * per_tile

        d0 = pltpu.make_async_copy(f0_hbm, ftab_v, sem_t)
        d1 = pltpu.make_async_copy(f1_hbm, ftab_v, sem_t)
        di = pltpu.make_async_copy(
            idx_hbm.at[pl.ds(base, per_tile)], idx_v, sem_i)
        di.start()

        @pl.when(col == 0)
        def _():
            d0.start()

        @pl.when(col == 1)
        def _():
            d1.start()

        d0.wait()
        di.wait()

        def body(i, carry):
            sl = pl.ds(i * lanes, lanes)
            vals_v[sl] = plsc.load_gather(ftab_v, [idx_v[sl]])
            return carry

        lax.fori_loop(0, per_tile // lanes, body, 0)

        @pl.when(col == 0)
        def _():
            pltpu.sync_copy(vals_v, o0_hbm.at[pl.ds(base, per_tile)])

        @pl.when(col == 1)
        def _():
            pltpu.sync_copy(vals_v, o1_hbm.at[pl.ds(base, per_tile)])

    return gather_kernel(idx1d, f0, f1)


def kernel(x, table, W1, b1, W2, b2):
    b, s = x.shape
    f0, f1 = _fuse_table(table, W1, b1, W2, b2)
    idx1d = x.astype(jnp.int32).reshape(-1)
    o0, o1 = _gather_rows(idx1d, f0, f1)
    return jnp.stack([o0, o1], axis=-1).reshape(b, s, W2.shape[1])


# speedup vs baseline: 1.0180x; 1.0021x over previous
import functools

import jax
import jax.numpy as jnp
from jax import lax
from jax.experimental import pallas as pl
from jax.experimental.pallas import tpu as pltpu
from jax.experimental.pallas import tpu_sc as plsc

_NC = 2
_NS = 16
_NW = _NC * _NS
_CHUNK = 128
_GROUP = 10


_NBUF = 4
_SUB = 2048


def _mlp(emb, w1_ref, b1_ref, w2t_ref, b2_ref):
    h = jnp.dot(emb, w1_ref[...], preferred_element_type=jnp.float32)
    h = jnp.maximum(h + b1_ref[...], 0.0)
    ot = lax.dot_general(w2t_ref[...], h,
                         dimension_numbers=(((1,), (1,)), ((), ())),
                         preferred_element_type=jnp.float32)
    return ot + b2_ref[...]


def _mlp_body(t_hbm, w1_ref, b1_ref, w2t_ref, b2_ref, o0_ref, o1_ref,
              tbuf, tailbuf, sems, tailsem):
    v = t_hbm.shape[0]
    n_full = v // _SUB
    tail = v - n_full * _SUB
    tail0 = n_full * _SUB

    def dma(j, slot):
        return pltpu.make_async_copy(
            t_hbm.at[pl.ds(j * _SUB, _SUB), :], tbuf.at[slot], sems.at[slot])

    tail_dma = pltpu.make_async_copy(
        t_hbm.at[pl.ds(tail0, tail), :], tailbuf, tailsem)
    tail_dma.start()
    for j in range(_NBUF):
        dma(j, j).start()

    def step(j, carry):
        slot = lax.rem(j, _NBUF)
        dma(j, slot).wait()
        ot = _mlp(tbuf[slot], w1_ref, b1_ref, w2t_ref, b2_ref)
        o0_ref[pl.ds(j * _SUB, _SUB)] = ot[0:1, :].reshape(_SUB)
        o1_ref[pl.ds(j * _SUB, _SUB)] = ot[1:2, :].reshape(_SUB)

        @pl.when(j + _NBUF < n_full)
        def _():
            dma(j + _NBUF, slot).start()

        return carry

    lax.fori_loop(0, n_full, step, 0)
    tail_dma.wait()
    ot = _mlp(tailbuf[...], w1_ref, b1_ref, w2t_ref, b2_ref)
    o0_ref[pl.ds(tail0, tail)] = ot[0:1, :].reshape(tail)
    o1_ref[pl.ds(tail0, tail)] = ot[1:2, :].reshape(tail)


def _fuse_table(table, W1, b1, W2, b2):
    v, d = table.shape
    dh = W1.shape[1]
    do = W2.shape[1]
    out1d = jax.ShapeDtypeStruct((v,), jnp.float32)
    return pl.pallas_call(
        _mlp_body,
        in_specs=[
            pl.BlockSpec(memory_space=pl.ANY),
            pl.BlockSpec((d, dh), lambda: (0, 0)),
            pl.BlockSpec((1, dh), lambda: (0, 0)),
            pl.BlockSpec((do, dh), lambda: (0, 0)),
            pl.BlockSpec((do, 1), lambda: (0, 0)),
        ],
        out_specs=[
            pl.BlockSpec((v,), lambda: (0,)),
            pl.BlockSpec((v,), lambda: (0,)),
        ],
        out_shape=[out1d, out1d],
        scratch_shapes=[
            pltpu.VMEM((_NBUF, _SUB, d), jnp.float32),
            pltpu.VMEM((v - (v // _SUB) * _SUB, d), jnp.float32),
            pltpu.SemaphoreType.DMA((_NBUF,)),
            pltpu.SemaphoreType.DMA,
        ],
    )(table, W1, b1.reshape(1, dh), W2.T, b2.reshape(do, 1))


def _gather_rows(idx1d, f0, f1):
    n_idx = idx1d.shape[0]
    v = f0.shape[0]
    per_tile = n_idx // _NS
    lanes = 16
    mesh = plsc.VectorSubcoreMesh(core_axis_name="c", subcore_axis_name="s")
    out1d = jax.ShapeDtypeStruct((n_idx,), jnp.float32)

    @functools.partial(
        pl.kernel,
        out_type=[out1d, out1d],
        mesh=mesh,
        scratch_types=[
            pltpu.VMEM((v,), jnp.float32),
            pltpu.VMEM((per_tile,), jnp.int32),
            pltpu.VMEM((per_tile,), jnp.float32),
            pltpu.SemaphoreType.DMA,
            pltpu.SemaphoreType.DMA,
        ],
        compiler_params=pltpu.CompilerParams(use_tc_tiling_on_sc=False, needs_layout_passes=False),
    )
    def gather_kernel(idx_hbm, f0_hbm, f1_hbm, o0_hbm, o1_hbm,
                      ftab_v, idx_v, vals_v, sem_t, sem_i):
        col = lax.axis_index("c")
        tile = lax.axis_index("s")
        base = tile * per_tile

        d0 = pltpu.make_async_copy(f0_hbm, ftab_v, sem_t)
        d1 = pltpu.make_async_copy(f1_hbm, ftab_v, sem_t)
        di = pltpu.make_async_copy(
            idx_hbm.at[pl.ds(base, per_tile)], idx_v, sem_i)
        di.start()

        @pl.when(col == 0)
        def _():
            d0.start()

        @pl.when(col == 1)
        def _():
            d1.start()

        d0.wait()
        di.wait()

        def body(i, carry):
            sl = pl.ds(i * lanes, lanes)
            vals_v[sl] = plsc.load_gather(ftab_v, [idx_v[sl]])
            return carry

        lax.fori_loop(0, per_tile // lanes, body, 0)

        @pl.when(col == 0)
        def _():
            pltpu.sync_copy(vals_v, o0_hbm.at[pl.ds(base, per_tile)])

        @pl.when(col == 1)
        def _():
            pltpu.sync_copy(vals_v, o1_hbm.at[pl.ds(base, per_tile)])

    return gather_kernel(idx1d, f0, f1)


def kernel(x, table, W1, b1, W2, b2):
    b, s = x.shape
    f0, f1 = _fuse_table(table, W1, b1, W2, b2)
    idx1d = x.astype(jnp.int32).reshape(-1)
    o0, o1 = _gather_rows(idx1d, f0, f1)
    return jnp.stack([o0.reshape(b, s), o1.reshape(b, s)], axis=-1)
